# Initial kernel scaffold; baseline (speedup 1.0000x reference)
import jax
import jax.numpy as jnp
from jax.experimental import pallas as pl


def kernel(x, tok_embed, pos_embed, ln_gamma, ln_beta):
    raise NotImplementedError("write your pallas kernel here")


# speedup vs baseline: 3.1955x; 3.1955x over previous
import functools

import jax
import jax.numpy as jnp
from jax import lax
from jax.experimental import pallas as pl
from jax.experimental.pallas import tpu as pltpu
from jax.experimental.pallas import tpu_sc as plsc

VOCAB = 100000
D = 128
SEQ = 200
BATCH = 4096
N_TOK = BATCH * SEQ
NVREG = D // 16

_info = plsc.get_sparse_core_info()
NC, NS = _info.num_cores, _info.num_subcores
NW = NC * NS
TOK_PER_W = N_TOK // NW
T = 128
NCHUNK = TOK_PER_W // T


def _rsqrt(v):
    vi = lax.bitcast_convert_type(v, jnp.int32)
    yi = jnp.int32(0x5F3759DF) - (vi >> 1)
    y = lax.bitcast_convert_type(yi, jnp.float32)
    half = 0.5 * v
    for _ in range(3):
        y = y * (1.5 - half * y * y)
    return y


def _tree_sum(vs):
    while len(vs) > 1:
        vs = [a + b for a, b in zip(vs[::2], vs[1::2])]
    return vs[0]


@functools.partial(
    pl.kernel,
    mesh=plsc.VectorSubcoreMesh(core_axis_name="c", subcore_axis_name="s"),
    compiler_params=pltpu.CompilerParams(needs_layout_passes=False),
    out_type=jax.ShapeDtypeStruct((N_TOK, D), jnp.float32),
    scratch_types=[
        pltpu.VMEM((T,), jnp.int32),
        pltpu.VMEM((T, D), jnp.float32),
        pltpu.VMEM((SEQ, D), jnp.float32),
        pltpu.VMEM((D,), jnp.float32),
        pltpu.VMEM((D,), jnp.float32),
        pltpu.SemaphoreType.DMA,
    ],
)
def _sc_embed_ln(x_hbm, tok_hbm, pos_hbm, gam_hbm, bet_hbm, out_hbm,
                 idx_v, rows_v, pos_v, gam_v, bet_v, sem):
    wid = lax.axis_index("s") * NC + lax.axis_index("c")

    pltpu.sync_copy(pos_hbm, pos_v)
    pltpu.sync_copy(gam_hbm, gam_v)
    pltpu.sync_copy(bet_hbm, bet_v)

    g = [gam_v[pl.ds(16 * j, 16)] for j in range(NVREG)]
    b = [bet_v[pl.ds(16 * j, 16)] for j in range(NVREG)]

    w_base = wid * TOK_PER_W

    def chunk_body(c, carry):
        base = w_base + c * T
        pltpu.sync_copy(x_hbm.at[pl.ds(base, T)], idx_v)
        pltpu.async_copy(tok_hbm.at[idx_v], rows_v, sem).wait()
        s0 = lax.rem(c * T, SEQ)

        def tok_body(t, carry2):
            s = lax.rem(s0 + t, SEQ)
            h = [rows_v[t, pl.ds(16 * j, 16)] + pos_v[s, pl.ds(16 * j, 16)]
                 for j in range(NVREG)]
            tot = jnp.sum(_tree_sum(h))
            totq = jnp.sum(_tree_sum([v * v for v in h]))
            mean = tot * (1.0 / D)
            var = totq * (1.0 / D) - mean * mean
            rstd = _rsqrt(var + 1e-5)
            mrs = mean * rstd
            for j in range(NVREG):
                rows_v[t, pl.ds(16 * j, 16)] = (h[j] * rstd - mrs) * g[j] + b[j]
            return carry2

        lax.fori_loop(0, T, tok_body, 0)
        pltpu.sync_copy(rows_v, out_hbm.at[pl.ds(base, T)])
        return carry

    lax.fori_loop(0, NCHUNK, chunk_body, 0)


def kernel(x, tok_embed, pos_embed, ln_gamma, ln_beta):
    x_flat = x.reshape(N_TOK)
    out = _sc_embed_ln(x_flat, tok_embed, pos_embed, ln_gamma, ln_beta)
    return out.reshape(BATCH, SEQ, D)


# speedup vs baseline: 11.0620x; 3.4617x over previous
import functools

import jax
import jax.numpy as jnp
from jax import lax
from jax.experimental import pallas as pl
from jax.experimental.pallas import tpu as pltpu
from jax.experimental.pallas import tpu_sc as plsc

VOCAB = 100000
D = 128
SEQ = 200
BATCH = 4096
N_TOK = BATCH * SEQ
NVREG = D // 16

_info = plsc.get_sparse_core_info()
NC, NS = _info.num_cores, _info.num_subcores
NW = NC * NS
TOK_PER_W = N_TOK // NW
T = 128
NCHUNK = TOK_PER_W // T
NBUF = 4
NGROUP = NCHUNK // NBUF
UNROLL = 4


def _rsqrt(v):
    vi = lax.bitcast_convert_type(v, jnp.int32)
    yi = jnp.int32(0x5F3759DF) - (vi >> 1)
    y = lax.bitcast_convert_type(yi, jnp.float32)
    half = 0.5 * v
    for _ in range(3):
        y = y * (1.5 - half * y * y)
    return y


def _tree_sum(vs):
    while len(vs) > 1:
        vs = [a + b for a, b in zip(vs[::2], vs[1::2])]
    return vs[0]


@functools.partial(
    pl.kernel,
    mesh=plsc.VectorSubcoreMesh(core_axis_name="c", subcore_axis_name="s"),
    compiler_params=pltpu.CompilerParams(needs_layout_passes=False),
    out_type=jax.ShapeDtypeStruct((N_TOK, D), jnp.float32),
    scratch_types=[
        pltpu.VMEM((NBUF, T), jnp.int32),
        pltpu.VMEM((NBUF, T, D), jnp.float32),
        pltpu.VMEM((SEQ, D), jnp.float32),
        pltpu.VMEM((D,), jnp.float32),
        pltpu.VMEM((D,), jnp.float32),
        pltpu.SemaphoreType.DMA,
        pltpu.SemaphoreType.DMA,
        pltpu.SemaphoreType.DMA,
        pltpu.SemaphoreType.DMA,
        pltpu.SemaphoreType.DMA,
        pltpu.SemaphoreType.DMA,
        pltpu.SemaphoreType.DMA,
        pltpu.SemaphoreType.DMA,
        pltpu.SemaphoreType.DMA,
        pltpu.SemaphoreType.DMA,
        pltpu.SemaphoreType.DMA,
        pltpu.SemaphoreType.DMA,
    ],
)
def _sc_embed_ln(x_hbm, tok_hbm, pos_hbm, gam_hbm, bet_hbm, out_hbm,
                 idx_v, rows_v, pos_v, gam_v, bet_v,
                 si0, si1, si2, si3, sg0, sg1, sg2, sg3, so0, so1, so2, so3):
    sem_i = [si0, si1, si2, si3]
    sem_g = [sg0, sg1, sg2, sg3]
    sem_o = [so0, so1, so2, so3]
    wid = lax.axis_index("s") * NC + lax.axis_index("c")
    w_base = wid * TOK_PER_W

    pltpu.sync_copy(pos_hbm, pos_v)
    pltpu.sync_copy(gam_hbm, gam_v)
    pltpu.sync_copy(bet_hbm, bet_v)

    g = [gam_v[pl.ds(16 * j, 16)] for j in range(NVREG)]
    b = [bet_v[pl.ds(16 * j, 16)] for j in range(NVREG)]

    def x_slice(c):
        return x_hbm.at[pl.ds(w_base + c * T, T)]

    def out_slice(c):
        return out_hbm.at[pl.ds(w_base + c * T, T)]

    def start_idx(c, p):
        pltpu.async_copy(x_slice(c), idx_v.at[p], sem_i[p])

    def drain_idx(c, p):
        pltpu.make_async_copy(x_slice(c), idx_v.at[p], sem_i[p]).wait()

    def start_gather(p):
        pltpu.async_copy(tok_hbm.at[idx_v.at[p]], rows_v.at[p], sem_g[p])

    def drain_gather(p):
        pltpu.make_async_copy(tok_hbm.at[idx_v.at[p]], rows_v.at[p],
                              sem_g[p]).wait()

    def start_out(c, p):
        pltpu.async_copy(rows_v.at[p], out_slice(c), sem_o[p])

    def drain_out(c, p):
        pltpu.make_async_copy(rows_v.at[p], out_slice(c), sem_o[p]).wait()

    start_idx(0, 0)
    start_idx(1, 1)
    start_idx(2, 2)
    drain_idx(0, 0)
    start_gather(0)
    drain_idx(1, 1)
    start_gather(1)

    def compute_chunk(c, p):
        s0 = lax.rem(c * T, SEQ)

        @plsc.parallel_loop(0, T, step=1, unroll=UNROLL)
        def _(t):
            sv = s0 + t
            s = jnp.where(sv >= SEQ, sv - SEQ, sv)
            h = [rows_v[p, t, pl.ds(16 * j, 16)] + pos_v[s, pl.ds(16 * j, 16)]
                 for j in range(NVREG)]
            tot = jnp.sum(_tree_sum(h))
            totq = jnp.sum(_tree_sum([v * v for v in h]))
            mean = tot * (1.0 / D)
            var = totq * (1.0 / D) - mean * mean
            rstd = _rsqrt(var + 1e-5)
            mrs = mean * rstd
            for j in range(NVREG):
                rows_v[p, t, pl.ds(16 * j, 16)] = \
                    (h[j] * rstd - mrs) * g[j] + b[j]

    def group_body(grp, carry):
        for bb in range(NBUF):
            c = grp * NBUF + bb

            @pl.when(c + 3 < NCHUNK)
            def _():
                start_idx(c + 3, (bb + 3) % NBUF)

            @pl.when(c + 2 < NCHUNK)
            def _():
                pf = (bb + 2) % NBUF
                drain_idx(c + 2, pf)

                @pl.when(c >= 2)
                def _():
                    drain_out(c - 2, pf)
                start_gather(pf)

            drain_gather(bb)
            compute_chunk(c, bb)
            start_out(c, bb)
        return carry

    lax.fori_loop(0, NGROUP, group_body, 0)
    for bb in range(NBUF):
        drain_out(NCHUNK - NBUF + bb, bb)


def kernel(x, tok_embed, pos_embed, ln_gamma, ln_beta):
    x_flat = x.reshape(N_TOK)
    out = _sc_embed_ln(x_flat, tok_embed, pos_embed, ln_gamma, ln_beta)
    return out.reshape(BATCH, SEQ, D)
